# baseline (speedup 1.0000x reference)
import functools

import jax
import jax.numpy as jnp
from jax import lax
from jax.experimental import pallas as pl
from jax.experimental.pallas import tpu as pltpu
from jax.experimental.pallas import tpu_sc as plsc

_NC = 2
_NS = 16
_NW = _NC * _NS
_B = 16384
_D = 64
_V = 1000000
_LANES = 16
_BLKW = 256
_NPAIR_FULL = _V // _BLKW
_KMAIN = 3904 // _NW
_NBUCK = 128
_NE = 2 * _B
_SENT = 127 << 24
_NROWS = _NE + 8192

_mesh = plsc.VectorSubcoreMesh(core_axis_name="c", subcore_axis_name="s")


def _lanes_iota():
    return lax.iota(jnp.int32, _LANES)


@functools.partial(
    pl.kernel,
    mesh=_mesh,
    out_type=pltpu.HBM((_NROWS, _D), jnp.float32),
    compiler_params=pltpu.CompilerParams(needs_layout_passes=False),
    scratch_types=[
        pltpu.VMEM((_NE + _LANES,), jnp.int32),
        pltpu.VMEM((_NE + _LANES,), jnp.int32),
        pltpu.VMEM((_D, _BLKW), jnp.float32),
        pltpu.VMEM((_D, _BLKW), jnp.float32),
        pltpu.VMEM((_LANES, _D), jnp.float32),
        pltpu.SMEM((_NBUCK,), jnp.int32),
        pltpu.SMEM((_NBUCK,), jnp.int32),
        pltpu.SMEM((_NBUCK,), jnp.int32),
        pltpu.SemaphoreType.DMA,
        pltpu.SemaphoreType.DMA,
        pltpu.SemaphoreType.DMA,
    ],
)
def _gather_rows(idx_hbm, nidx_hbm, table_hbm, tail_hbm, rows_hbm,
                 ids_v, hit_v, blk0, blk1, stage,
                 cnt_s, off_s, cur_s, sem0, sem1, semo):
    wid = lax.axis_index("s") * _NC + lax.axis_index("c")
    _iota = _lanes_iota()
    _lane0 = _iota == 0
    pltpu.sync_copy(idx_hbm, ids_v.at[pl.ds(0, _B)])
    pltpu.sync_copy(nidx_hbm, ids_v.at[pl.ds(_B, _B)])

    def fbody(g, hn):
        rv = ids_v[pl.ds(g * _LANES, _LANES)]
        blk = rv >> 8
        own = (blk & (_NW - 1)) == wid
        pk = ((blk >> 5) << 24) | ((rv & 255) << 16) | (g * _LANES + _iota)
        n = plsc.all_reduce_population_count(own)[0]
        plsc.store_compressed(hit_v.at[pl.ds(hn, _LANES)], pk, mask=own)
        return hn + n

    hn = lax.fori_loop(0, _NE // _LANES, fbody, jnp.int32(0))
    hit_v[pl.ds(hn, _LANES)] = jnp.full((_LANES,), _SENT, jnp.int32)

    def zbody(i, _):
        cnt_s[i] = 0
        return 0

    lax.fori_loop(0, _NBUCK, zbody, 0)
    ng = (hn + _LANES - 1) // _LANES

    def cbody(t, _):
        kv = hit_v[pl.ds(t * _LANES, _LANES)] >> 24
        for j in range(_LANES):
            k = kv[j]
            cnt_s[k] = cnt_s[k] + 1
        return 0

    lax.fori_loop(0, ng, cbody, 0)

    def pbody(i, s):
        off_s[i] = s
        cur_s[i] = s
        return s + cnt_s[i]

    lax.fori_loop(0, _NBUCK, pbody, jnp.int32(0))

    def sbody(t, _):
        hv = hit_v[pl.ds(t * _LANES, _LANES)]
        kv = hv >> 24
        for j in range(_LANES):
            k = kv[j]
            c = cur_s[k]
            cur_s[k] = c + 1
            plsc.store_scatter(ids_v, [jnp.full((_LANES,), c, jnp.int32)],
                               jnp.full((_LANES,), hv[j], jnp.int32),
                               mask=_lane0)
        return 0

    lax.fori_loop(0, ng, sbody, 0)

    def issue(k, buf, sem):
        col0 = pl.multiple_of((k * _NW + wid) * _BLKW, _BLKW)
        pltpu.async_copy(table_hbm.at[pl.ds(0, _D), pl.ds(col0, _BLKW)],
                         buf, sem)

    def drain(buf, sem):
        pltpu.make_async_copy(table_hbm.at[pl.ds(0, _D), pl.ds(0, _BLKW)],
                              buf, sem).wait()

    def process(k, buf, lane_off=0):
        start = off_s[k]
        cnt = cnt_s[k]
        ngr = (cnt + _LANES - 1) // _LANES

        def gbody(t, _):
            rem = cnt - t * _LANES
            hv = ids_v[pl.ds(start + t * _LANES, _LANES)]
            lv = jnp.maximum(((hv >> 16) & 255) - lane_off, 0)
            for c in range(_D):
                g = plsc.load_gather(buf, [jnp.full((_LANES,), c, jnp.int32),
                                           lv])
                plsc.store_scatter(stage, [_iota,
                                           jnp.full((_LANES,), c, jnp.int32)],
                                   g)
            for j in range(_LANES):
                @pl.when(j < rem)
                def _():
                    pltpu.async_copy(stage.at[j], rows_hbm.at[hv[j] & 0x7FFF],
                                     semo)
            return 0

        lax.fori_loop(0, ngr, gbody, 0)

    issue(0, blk0, sem0)
    issue(1, blk1, sem1)

    def body(h, _):
        k0 = 2 * h
        drain(blk0, sem0)
        process(k0, blk0)

        @pl.when(k0 + 2 < _KMAIN)
        def _():
            issue(k0 + 2, blk0, sem0)

        drain(blk1, sem1)
        process(k0 + 1, blk1)

        @pl.when(k0 + 3 < _KMAIN)
        def _():
            issue(k0 + 3, blk1, sem1)

        return 0

    lax.fori_loop(0, _KMAIN // 2, body, 0)

    @pl.when(wid < 2)
    def _():
        col0 = pl.multiple_of((3904 + wid) * _BLKW, _BLKW)
        pltpu.sync_copy(table_hbm.at[pl.ds(0, _D), pl.ds(col0, _BLKW)], blk0)
        process(_KMAIN, blk0)

    @pl.when(wid == 2)
    def _():
        pltpu.sync_copy(tail_hbm, blk0)
        process(_KMAIN, blk0)

    def dbody16(i, _):
        pltpu.make_async_copy(rows_hbm.at[pl.ds(0, _LANES)], stage, semo).wait()
        return 0

    lax.fori_loop(0, hn >> 4, dbody16, 0)

    def dbody1(i, _):
        pltpu.make_async_copy(rows_hbm.at[0], stage.at[0], semo).wait()
        return 0

    lax.fori_loop(0, hn & (_LANES - 1), dbody1, 0)


_BPW = _B // _NW
_PCH = 128


@functools.partial(
    pl.kernel,
    mesh=_mesh,
    out_type=jax.ShapeDtypeStruct((_NW, _LANES), jnp.float32),
    compiler_params=pltpu.CompilerParams(needs_layout_passes=False),
    scratch_types=[
        pltpu.VMEM((_PCH, _D), jnp.float32),
        pltpu.VMEM((_PCH, _D), jnp.float32),
        pltpu.VMEM((_LANES,), jnp.float32),
        pltpu.SemaphoreType.DMA,
        pltpu.SemaphoreType.DMA,
    ],
)
def _pair_reduce(rows_hbm, out_hbm, a_v, b_v, acc_v, sem_a, sem_b):
    wid = lax.axis_index("s") * _NC + lax.axis_index("c")
    base = wid * _BPW

    def issue(t):
        pltpu.async_copy(rows_hbm.at[pl.ds(base + t * _PCH, _PCH)], a_v, sem_a)
        pltpu.async_copy(rows_hbm.at[pl.ds(_B + base + t * _PCH, _PCH)], b_v,
                         sem_b)

    def wait():
        pltpu.make_async_copy(rows_hbm.at[pl.ds(0, _PCH)], a_v, sem_a).wait()
        pltpu.make_async_copy(rows_hbm.at[pl.ds(0, _PCH)], b_v, sem_b).wait()

    def chunk(t, acc):
        issue(t)
        wait()

        def body(i, acc):
            for j in range(_D // _LANES):
                a = a_v[i, pl.ds(j * _LANES, _LANES)]
                b = b_v[i, pl.ds(j * _LANES, _LANES)]
                d = a - b
                acc = acc + d * d
            return acc

        return lax.fori_loop(0, _PCH, body, acc)

    acc = lax.fori_loop(0, _BPW // _PCH, chunk,
                        jnp.zeros((_LANES,), jnp.float32))
    acc_v[...] = acc
    pltpu.sync_copy(acc_v, out_hbm.at[wid])


def kernel(index_vec, neighbor_index_vec, non_neighbor1, non_neighbor2,
           radius_sum, radius_sum2, radius_sum3, exist_non_neighbor,
           emb_weight):
    idx = index_vec.astype(jnp.int32)
    nidx = neighbor_index_vec.astype(jnp.int32)
    table_t = emb_weight.T
    tail_t = jnp.pad(table_t[:, 3906 * _BLKW:],
                     ((0, 0), (0, 3907 * _BLKW - _V)))
    rows = _gather_rows(idx, nidx, table_t, tail_t)
    partials = _pair_reduce(rows)
    dist = jnp.sqrt(jnp.sum(partials))
    l1 = jnp.abs(dist - radius_sum)
    return jnp.where(dist - radius_sum < 0, 10.0 * l1, l1)


# speedup vs baseline: 1.0054x; 1.0054x over previous
import functools

import jax
import jax.numpy as jnp
from jax import lax
from jax.experimental import pallas as pl
from jax.experimental.pallas import tpu as pltpu
from jax.experimental.pallas import tpu_sc as plsc

_NC = 2
_NS = 16
_NW = _NC * _NS
_B = 16384
_D = 64
_V = 1000000
_LANES = 16
_BLKW = 256
_NPAIR_FULL = _V // _BLKW
_KMAIN = 3904 // _NW
_NBUCK = 128
_NE = 2 * _B
_SENT = 127 << 24
_NROWS = _NE + 8192

_mesh = plsc.VectorSubcoreMesh(core_axis_name="c", subcore_axis_name="s")


def _lanes_iota():
    return lax.iota(jnp.int32, _LANES)


@functools.partial(
    pl.kernel,
    mesh=_mesh,
    out_type=pltpu.HBM((_NROWS, _D), jnp.float32),
    compiler_params=pltpu.CompilerParams(needs_layout_passes=False),
    scratch_types=[
        pltpu.VMEM((_NE + _LANES,), jnp.int32),
        pltpu.VMEM((_NE + _LANES,), jnp.int32),
        pltpu.VMEM((_D, _BLKW), jnp.float32),
        pltpu.VMEM((_D, _BLKW), jnp.float32),
        pltpu.VMEM((_LANES, _D), jnp.float32),
        pltpu.SMEM((_NBUCK,), jnp.int32),
        pltpu.SMEM((_NBUCK,), jnp.int32),
        pltpu.SMEM((_NBUCK,), jnp.int32),
        pltpu.SemaphoreType.DMA,
        pltpu.SemaphoreType.DMA,
        pltpu.SemaphoreType.DMA,
    ],
)
def _gather_rows(idx_hbm, nidx_hbm, table_hbm, tail_hbm, rows_hbm,
                 ids_v, hit_v, blk0, blk1, stage,
                 cnt_s, off_s, cur_s, sem0, sem1, semo):
    wid = lax.axis_index("s") * _NC + lax.axis_index("c")
    _iota = _lanes_iota()
    _lane0 = _iota == 0
    pltpu.sync_copy(idx_hbm, ids_v.at[pl.ds(0, _B)])
    pltpu.sync_copy(nidx_hbm, ids_v.at[pl.ds(_B, _B)])

    def issue(k, buf, sem):
        col0 = pl.multiple_of((k * _NW + wid) * _BLKW, _BLKW)
        pltpu.async_copy(table_hbm.at[pl.ds(0, _D), pl.ds(col0, _BLKW)],
                         buf, sem)

    def drain(buf, sem):
        pltpu.make_async_copy(table_hbm.at[pl.ds(0, _D), pl.ds(0, _BLKW)],
                              buf, sem).wait()

    issue(0, blk0, sem0)
    issue(1, blk1, sem1)

    def fbody(g, hn):
        rv = ids_v[pl.ds(g * _LANES, _LANES)]
        blk = rv >> 8
        own = (blk & (_NW - 1)) == wid
        pk = ((blk >> 5) << 24) | ((rv & 255) << 16) | (g * _LANES + _iota)
        n = plsc.all_reduce_population_count(own)[0]
        plsc.store_compressed(hit_v.at[pl.ds(hn, _LANES)], pk, mask=own)
        return hn + n

    hn = lax.fori_loop(0, _NE // _LANES, fbody, jnp.int32(0))
    hit_v[pl.ds(hn, _LANES)] = jnp.full((_LANES,), _SENT, jnp.int32)

    def zbody(i, _):
        cnt_s[i] = 0
        return 0

    lax.fori_loop(0, _NBUCK, zbody, 0)
    ng = (hn + _LANES - 1) // _LANES

    def cbody(t, _):
        kv = hit_v[pl.ds(t * _LANES, _LANES)] >> 24
        for j in range(_LANES):
            k = kv[j]
            cnt_s[k] = cnt_s[k] + 1
        return 0

    lax.fori_loop(0, ng, cbody, 0)

    def pbody(i, s):
        off_s[i] = s
        cur_s[i] = s
        return s + cnt_s[i]

    lax.fori_loop(0, _NBUCK, pbody, jnp.int32(0))

    def sbody(t, _):
        hv = hit_v[pl.ds(t * _LANES, _LANES)]
        kv = hv >> 24
        for j in range(_LANES):
            k = kv[j]
            c = cur_s[k]
            cur_s[k] = c + 1
            plsc.store_scatter(ids_v, [jnp.full((_LANES,), c, jnp.int32)],
                               jnp.full((_LANES,), hv[j], jnp.int32),
                               mask=_lane0)
        return 0

    lax.fori_loop(0, ng, sbody, 0)

    def process(k, buf, lane_off=0):
        start = off_s[k]
        cnt = cnt_s[k]
        ngr = (cnt + _LANES - 1) // _LANES

        def gbody(t, _):
            rem = cnt - t * _LANES
            hv = ids_v[pl.ds(start + t * _LANES, _LANES)]
            lv = jnp.maximum(((hv >> 16) & 255) - lane_off, 0)
            for c in range(_D):
                g = plsc.load_gather(buf, [jnp.full((_LANES,), c, jnp.int32),
                                           lv])
                plsc.store_scatter(stage, [_iota,
                                           jnp.full((_LANES,), c, jnp.int32)],
                                   g)
            for j in range(_LANES):
                @pl.when(j < rem)
                def _():
                    pltpu.async_copy(stage.at[j], rows_hbm.at[hv[j] & 0x7FFF],
                                     semo)
            return 0

        lax.fori_loop(0, ngr, gbody, 0)

    def body(h, _):
        k0 = 2 * h
        drain(blk0, sem0)
        process(k0, blk0)

        @pl.when(k0 + 2 < _KMAIN)
        def _():
            issue(k0 + 2, blk0, sem0)

        drain(blk1, sem1)
        process(k0 + 1, blk1)

        @pl.when(k0 + 3 < _KMAIN)
        def _():
            issue(k0 + 3, blk1, sem1)

        return 0

    lax.fori_loop(0, _KMAIN // 2, body, 0)

    @pl.when(wid < 2)
    def _():
        col0 = pl.multiple_of((3904 + wid) * _BLKW, _BLKW)
        pltpu.sync_copy(table_hbm.at[pl.ds(0, _D), pl.ds(col0, _BLKW)], blk0)
        process(_KMAIN, blk0)

    @pl.when(wid == 2)
    def _():
        pltpu.sync_copy(tail_hbm, blk0)
        process(_KMAIN, blk0)

    def dbody16(i, _):
        pltpu.make_async_copy(rows_hbm.at[pl.ds(0, _LANES)], stage, semo).wait()
        return 0

    lax.fori_loop(0, hn >> 4, dbody16, 0)

    def dbody1(i, _):
        pltpu.make_async_copy(rows_hbm.at[0], stage.at[0], semo).wait()
        return 0

    lax.fori_loop(0, hn & (_LANES - 1), dbody1, 0)


_BPW = _B // _NW
_PCH = 128


@functools.partial(
    pl.kernel,
    mesh=_mesh,
    out_type=jax.ShapeDtypeStruct((_NW, _LANES), jnp.float32),
    compiler_params=pltpu.CompilerParams(needs_layout_passes=False),
    scratch_types=[
        pltpu.VMEM((_PCH, _D), jnp.float32),
        pltpu.VMEM((_PCH, _D), jnp.float32),
        pltpu.VMEM((_LANES,), jnp.float32),
        pltpu.SemaphoreType.DMA,
        pltpu.SemaphoreType.DMA,
    ],
)
def _pair_reduce(rows_hbm, out_hbm, a_v, b_v, acc_v, sem_a, sem_b):
    wid = lax.axis_index("s") * _NC + lax.axis_index("c")
    base = wid * _BPW

    def issue(t):
        pltpu.async_copy(rows_hbm.at[pl.ds(base + t * _PCH, _PCH)], a_v, sem_a)
        pltpu.async_copy(rows_hbm.at[pl.ds(_B + base + t * _PCH, _PCH)], b_v,
                         sem_b)

    def wait():
        pltpu.make_async_copy(rows_hbm.at[pl.ds(0, _PCH)], a_v, sem_a).wait()
        pltpu.make_async_copy(rows_hbm.at[pl.ds(0, _PCH)], b_v, sem_b).wait()

    def chunk(t, acc):
        issue(t)
        wait()

        def body(i, acc):
            for j in range(_D // _LANES):
                a = a_v[i, pl.ds(j * _LANES, _LANES)]
                b = b_v[i, pl.ds(j * _LANES, _LANES)]
                d = a - b
                acc = acc + d * d
            return acc

        return lax.fori_loop(0, _PCH, body, acc)

    acc = lax.fori_loop(0, _BPW // _PCH, chunk,
                        jnp.zeros((_LANES,), jnp.float32))
    acc_v[...] = acc
    pltpu.sync_copy(acc_v, out_hbm.at[wid])


def kernel(index_vec, neighbor_index_vec, non_neighbor1, non_neighbor2,
           radius_sum, radius_sum2, radius_sum3, exist_non_neighbor,
           emb_weight):
    idx = index_vec.astype(jnp.int32)
    nidx = neighbor_index_vec.astype(jnp.int32)
    table_t = emb_weight.T
    tail_t = jnp.pad(table_t[:, 3906 * _BLKW:],
                     ((0, 0), (0, 3907 * _BLKW - _V)))
    rows = _gather_rows(idx, nidx, table_t, tail_t)
    partials = _pair_reduce(rows)
    dist = jnp.sqrt(jnp.sum(partials))
    l1 = jnp.abs(dist - radius_sum)
    return jnp.where(dist - radius_sum < 0, 10.0 * l1, l1)


# speedup vs baseline: 1.1831x; 1.1767x over previous
import functools

import jax
import jax.numpy as jnp
from jax import lax
from jax.experimental import pallas as pl
from jax.experimental.pallas import tpu as pltpu
from jax.experimental.pallas import tpu_sc as plsc

_NC = 2
_NS = 16
_NW = _NC * _NS
_B = 16384
_D = 64
_V = 1000000
_LANES = 16
_BLKW = 256
_NPAIR_FULL = _V // _BLKW
_KMAIN = 3904 // _NW
_NBUCK = 128
_NE = 2 * _B
_SENT = 127 << 24
_NROWS = _NE + 8192

_mesh = plsc.VectorSubcoreMesh(core_axis_name="c", subcore_axis_name="s")


def _lanes_iota():
    return lax.iota(jnp.int32, _LANES)


@functools.partial(
    pl.kernel,
    mesh=_mesh,
    out_type=pltpu.HBM((_NROWS, _D), jnp.float32),
    compiler_params=pltpu.CompilerParams(needs_layout_passes=False),
    scratch_types=[
        pltpu.VMEM((_NE + _LANES,), jnp.int32),
        pltpu.VMEM((_NE + _LANES,), jnp.int32),
        pltpu.VMEM((_D, _BLKW), jnp.float32),
        pltpu.VMEM((_D, _BLKW), jnp.float32),
        pltpu.VMEM((_D, _BLKW), jnp.float32),
        pltpu.VMEM((_LANES, _D), jnp.float32),
        pltpu.SMEM((_NBUCK,), jnp.int32),
        pltpu.SMEM((_NBUCK,), jnp.int32),
        pltpu.SMEM((_NBUCK,), jnp.int32),
        pltpu.SemaphoreType.DMA,
        pltpu.SemaphoreType.DMA,
        pltpu.SemaphoreType.DMA,
        pltpu.SemaphoreType.DMA,
    ],
)
def _gather_rows(idx_hbm, nidx_hbm, table_hbm, tail_hbm, rows_hbm,
                 ids_v, hit_v, blk0, blk1, blk2, stage,
                 cnt_s, off_s, cur_s, sem0, sem1, sem2, semo):
    wid = lax.axis_index("s") * _NC + lax.axis_index("c")
    _iota = _lanes_iota()
    _lane0 = _iota == 0
    pltpu.sync_copy(idx_hbm, ids_v.at[pl.ds(0, _B)])
    pltpu.sync_copy(nidx_hbm, ids_v.at[pl.ds(_B, _B)])

    def issue(k, buf, sem):
        col0 = pl.multiple_of((k * _NW + wid) * _BLKW, _BLKW)
        pltpu.async_copy(table_hbm.at[pl.ds(0, _D), pl.ds(col0, _BLKW)],
                         buf, sem)

    def drain(buf, sem):
        pltpu.make_async_copy(table_hbm.at[pl.ds(0, _D), pl.ds(0, _BLKW)],
                              buf, sem).wait()

    issue(0, blk0, sem0)
    issue(1, blk1, sem1)
    issue(2, blk2, sem2)

    def fbody(g, hn):
        rv = ids_v[pl.ds(g * _LANES, _LANES)]
        blk = rv >> 8
        own = (blk & (_NW - 1)) == wid
        pk = ((blk >> 5) << 24) | ((rv & 255) << 16) | (g * _LANES + _iota)
        n = plsc.all_reduce_population_count(own)[0]
        plsc.store_compressed(hit_v.at[pl.ds(hn, _LANES)], pk, mask=own)
        return hn + n

    hn = lax.fori_loop(0, _NE // _LANES, fbody, jnp.int32(0))
    hit_v[pl.ds(hn, _LANES)] = jnp.full((_LANES,), _SENT, jnp.int32)

    def zbody(i, _):
        cnt_s[i] = 0
        return 0

    lax.fori_loop(0, _NBUCK, zbody, 0)
    ng = (hn + _LANES - 1) // _LANES

    def cbody(t, _):
        kv = hit_v[pl.ds(t * _LANES, _LANES)] >> 24
        for j in range(_LANES):
            k = kv[j]
            cnt_s[k] = cnt_s[k] + 1
        return 0

    lax.fori_loop(0, ng, cbody, 0)

    def pbody(i, s):
        off_s[i] = s
        cur_s[i] = s
        return s + cnt_s[i]

    lax.fori_loop(0, _NBUCK, pbody, jnp.int32(0))

    def sbody(t, _):
        hv = hit_v[pl.ds(t * _LANES, _LANES)]
        kv = hv >> 24
        for j in range(_LANES):
            k = kv[j]
            c = cur_s[k]
            cur_s[k] = c + 1
            plsc.store_scatter(ids_v, [jnp.full((_LANES,), c, jnp.int32)],
                               jnp.full((_LANES,), hv[j], jnp.int32),
                               mask=_lane0)
        return 0

    lax.fori_loop(0, ng, sbody, 0)

    def process(k, buf, lane_off=0):
        start = off_s[k]
        cnt = cnt_s[k]
        ngr = (cnt + _LANES - 1) // _LANES

        def gbody(t, _):
            rem = cnt - t * _LANES
            hv = ids_v[pl.ds(start + t * _LANES, _LANES)]
            lv = jnp.maximum(((hv >> 16) & 255) - lane_off, 0)
            for c in range(_D):
                g = plsc.load_gather(buf, [jnp.full((_LANES,), c, jnp.int32),
                                           lv])
                plsc.store_scatter(stage, [_iota,
                                           jnp.full((_LANES,), c, jnp.int32)],
                                   g)
            for j in range(_LANES):
                @pl.when(j < rem)
                def _():
                    pltpu.async_copy(stage.at[j], rows_hbm.at[hv[j] & 0x7FFF],
                                     semo)
            return 0

        lax.fori_loop(0, ngr, gbody, 0)

    def body(h, _):
        k0 = 3 * h
        drain(blk0, sem0)
        process(k0, blk0)

        @pl.when(k0 + 3 < _KMAIN)
        def _():
            issue(k0 + 3, blk0, sem0)

        drain(blk1, sem1)
        process(k0 + 1, blk1)

        @pl.when(k0 + 4 < _KMAIN)
        def _():
            issue(k0 + 4, blk1, sem1)

        drain(blk2, sem2)
        process(k0 + 2, blk2)

        @pl.when(k0 + 5 < _KMAIN)
        def _():
            issue(k0 + 5, blk2, sem2)

        return 0

    lax.fori_loop(0, 40, body, 0)
    drain(blk0, sem0)
    process(120, blk0)
    drain(blk1, sem1)
    process(121, blk1)

    @pl.when(wid < 2)
    def _():
        col0 = pl.multiple_of((3904 + wid) * _BLKW, _BLKW)
        pltpu.sync_copy(table_hbm.at[pl.ds(0, _D), pl.ds(col0, _BLKW)], blk0)
        process(_KMAIN, blk0)

    @pl.when(wid == 2)
    def _():
        pltpu.sync_copy(tail_hbm, blk0)
        process(_KMAIN, blk0)

    def dbody16(i, _):
        pltpu.make_async_copy(rows_hbm.at[pl.ds(0, _LANES)], stage, semo).wait()
        return 0

    lax.fori_loop(0, hn >> 4, dbody16, 0)

    def dbody1(i, _):
        pltpu.make_async_copy(rows_hbm.at[0], stage.at[0], semo).wait()
        return 0

    lax.fori_loop(0, hn & (_LANES - 1), dbody1, 0)


_BPW = _B // _NW
_PCH = 128


@functools.partial(
    pl.kernel,
    mesh=_mesh,
    out_type=jax.ShapeDtypeStruct((_NW, _LANES), jnp.float32),
    compiler_params=pltpu.CompilerParams(needs_layout_passes=False),
    scratch_types=[
        pltpu.VMEM((_PCH, _D), jnp.float32),
        pltpu.VMEM((_PCH, _D), jnp.float32),
        pltpu.VMEM((_LANES,), jnp.float32),
        pltpu.SemaphoreType.DMA,
        pltpu.SemaphoreType.DMA,
    ],
)
def _pair_reduce(rows_hbm, out_hbm, a_v, b_v, acc_v, sem_a, sem_b):
    wid = lax.axis_index("s") * _NC + lax.axis_index("c")
    base = wid * _BPW

    def issue(t):
        pltpu.async_copy(rows_hbm.at[pl.ds(base + t * _PCH, _PCH)], a_v, sem_a)
        pltpu.async_copy(rows_hbm.at[pl.ds(_B + base + t * _PCH, _PCH)], b_v,
                         sem_b)

    def wait():
        pltpu.make_async_copy(rows_hbm.at[pl.ds(0, _PCH)], a_v, sem_a).wait()
        pltpu.make_async_copy(rows_hbm.at[pl.ds(0, _PCH)], b_v, sem_b).wait()

    def chunk(t, acc):
        issue(t)
        wait()

        def body(i, acc):
            for j in range(_D // _LANES):
                a = a_v[i, pl.ds(j * _LANES, _LANES)]
                b = b_v[i, pl.ds(j * _LANES, _LANES)]
                d = a - b
                acc = acc + d * d
            return acc

        return lax.fori_loop(0, _PCH, body, acc)

    acc = lax.fori_loop(0, _BPW // _PCH, chunk,
                        jnp.zeros((_LANES,), jnp.float32))
    acc_v[...] = acc
    pltpu.sync_copy(acc_v, out_hbm.at[wid])


def kernel(index_vec, neighbor_index_vec, non_neighbor1, non_neighbor2,
           radius_sum, radius_sum2, radius_sum3, exist_non_neighbor,
           emb_weight):
    idx = index_vec.astype(jnp.int32)
    nidx = neighbor_index_vec.astype(jnp.int32)
    table_t = emb_weight.T
    tail_t = jnp.pad(table_t[:, 3906 * _BLKW:],
                     ((0, 0), (0, 3907 * _BLKW - _V)))
    rows = _gather_rows(idx, nidx, table_t, tail_t)
    partials = _pair_reduce(rows)
    dist = jnp.sqrt(jnp.sum(partials))
    l1 = jnp.abs(dist - radius_sum)
    return jnp.where(dist - radius_sum < 0, 10.0 * l1, l1)
